# baseline (speedup 1.0000x reference)
import jax
import jax.numpy as jnp
from jax.experimental import pallas as pl
from jax.experimental.pallas import tpu as pltpu

_G = 4


def _vq_kernel(x_ref, embed_ref, q_ref, ind_ref,
               embed_t2_ref, neg_e2_ref):
    @pl.when(pl.program_id(0) == 0)
    def _prep():
        embed = embed_ref[...]
        et2 = 2.0 * embed.T
        embed_t2_ref[...] = et2
        neg_e2_ref[...] = -0.25 * jnp.sum(et2 * et2, axis=1, keepdims=True)

    embed_t2 = embed_t2_ref[...]
    neg_e2 = neg_e2_ref[...]
    embed = embed_ref[...]

    for g in range(_G):
        x = x_ref[g]

        s = jnp.dot(embed_t2, x, preferred_element_type=jnp.float32)
        neg = s + neg_e2

        idx = jnp.argmax(neg, axis=0).astype(jnp.int32)

        iota_k = jax.lax.broadcasted_iota(jnp.int32, neg.shape, 0)
        onehot = (iota_k == idx[None, :]).astype(jnp.float32)
        q = jnp.dot(embed, onehot, preferred_element_type=jnp.float32)

        q_ref[g] = q
        ind_ref[g, 0] = idx


def kernel(input, embed):
    b, c, h, w = input.shape
    n_codes = embed.shape[1]
    p = h * w

    x = input.reshape(b, c, p)

    q, ind = pl.pallas_call(
        _vq_kernel,
        grid=(b // _G,),
        in_specs=[
            pl.BlockSpec((_G, c, p), lambda i: (i, 0, 0)),
            pl.BlockSpec((c, n_codes), lambda i: (0, 0)),
        ],
        out_specs=[
            pl.BlockSpec((_G, c, p), lambda i: (i, 0, 0)),
            pl.BlockSpec((_G, 1, p), lambda i: (i, 0, 0)),
        ],
        out_shape=[
            jax.ShapeDtypeStruct((b, c, p), jnp.float32),
            jax.ShapeDtypeStruct((b, 1, p), jnp.int32),
        ],
        scratch_shapes=[
            pltpu.VMEM((n_codes, c), jnp.float32),
            pltpu.VMEM((n_codes, 1), jnp.float32),
        ],
    )(x, embed)

    embed_ind = ind.reshape(b, h, w)
    return ((q * 1.0).reshape(b, c, h, w), q.reshape(b, c, h, w), embed_ind)


# speedup vs baseline: 1.0865x; 1.0865x over previous
import jax
import jax.numpy as jnp
from jax.experimental import pallas as pl
from jax.experimental.pallas import tpu as pltpu

_G = 4


def _vq_kernel(x_ref, embed_ref, qwg_ref, q_ref, ind_ref,
               embed_t2_ref, neg_e2_ref):
    @pl.when(pl.program_id(0) == 0)
    def _prep():
        embed = embed_ref[...]
        et2 = 2.0 * embed.T
        embed_t2_ref[...] = et2
        neg_e2_ref[...] = -0.25 * jnp.sum(et2 * et2, axis=1, keepdims=True)

    embed_t2 = embed_t2_ref[...]
    neg_e2 = neg_e2_ref[...]
    embed = embed_ref[...]

    for g in range(_G):
        x = x_ref[g]

        s = jnp.dot(embed_t2, x, preferred_element_type=jnp.float32)
        neg = s + neg_e2

        idx = jnp.argmax(neg, axis=0).astype(jnp.int32)

        iota_k = jax.lax.broadcasted_iota(jnp.int32, neg.shape, 0)
        onehot = (iota_k == idx[None, :]).astype(jnp.float32)
        q = jnp.dot(embed, onehot, preferred_element_type=jnp.float32)

        qwg_ref[g] = q
        q_ref[g] = q
        ind_ref[g, 0] = idx


def kernel(input, embed):
    b, c, h, w = input.shape
    n_codes = embed.shape[1]
    p = h * w

    x = input.reshape(b, c, p)

    qwg, q, ind = pl.pallas_call(
        _vq_kernel,
        grid=(b // _G,),
        in_specs=[
            pl.BlockSpec((_G, c, p), lambda i: (i, 0, 0)),
            pl.BlockSpec((c, n_codes), lambda i: (0, 0)),
        ],
        out_specs=[
            pl.BlockSpec((_G, c, p), lambda i: (i, 0, 0)),
            pl.BlockSpec((_G, c, p), lambda i: (i, 0, 0)),
            pl.BlockSpec((_G, 1, p), lambda i: (i, 0, 0)),
        ],
        out_shape=[
            jax.ShapeDtypeStruct((b, c, p), jnp.float32),
            jax.ShapeDtypeStruct((b, c, p), jnp.float32),
            jax.ShapeDtypeStruct((b, 1, p), jnp.int32),
        ],
        scratch_shapes=[
            pltpu.VMEM((n_codes, c), jnp.float32),
            pltpu.VMEM((n_codes, 1), jnp.float32),
        ],
    )(x, embed)

    embed_ind = ind.reshape(b, h, w)
    return (qwg.reshape(b, c, h, w), q.reshape(b, c, h, w), embed_ind)
